# baseline (speedup 1.0000x reference)
import functools
import jax
import jax.numpy as jnp
from jax.experimental import pallas as pl

_H = 16
_TOPK = 64
_TEMPERATURE = 1.0
_BQ = 2048


def _qkv_body(x_ref, w_ref, b_ref, out_ref):
    x = x_ref[...]
    w = w_ref[0]
    b = b_ref[0]
    out_ref[0] = jnp.dot(x, w, preferred_element_type=jnp.float32) + b[0][None, :]


def _attn_body(q_ref, kt_ref, v_ref, o_ref, *, topk, scale):
    q = q_ref[0]
    kt = kt_ref[0]
    v = v_ref[0]
    s = jnp.dot(q, kt, preferred_element_type=jnp.float32) * scale

    def to_f32(k):
        return jax.lax.bitcast_convert_type(
            k ^ ((k >> 31) & jnp.int32(0x7FFFFFFF)), jnp.float32
        )

    bq = s.shape[0]
    lo0 = jnp.full((bq, 1), jnp.int32(-2139095041), jnp.int32)
    hi0 = jnp.full((bq, 1), jnp.int32(0x7F800000), jnp.int32)

    def avg(a, b):
        return (a >> 1) + (b >> 1) + (a & b & 1)

    def count_ge(thr):
        return jnp.sum((s >= to_f32(thr)).astype(jnp.int32), axis=1, keepdims=True)

    def step4(_, carry):
        lo, hi = carry
        m2 = avg(lo, hi)
        m1 = avg(lo, m2)
        m3 = avg(m2, hi)
        c1 = count_ge(m1) >= topk
        c2 = count_ge(m2) >= topk
        c3 = count_ge(m3) >= topk
        lo = jnp.where(c3, m3, jnp.where(c2, m2, jnp.where(c1, m1, lo)))
        hi = jnp.where(c3, hi, jnp.where(c2, m3, jnp.where(c1, m2, m1)))
        return lo, hi

    def step(_, carry):
        lo, hi = carry
        mid = avg(lo, hi)
        ge = count_ge(mid) >= topk
        return jnp.where(ge, mid, lo), jnp.where(ge, hi, mid)

    lo, hi = jax.lax.fori_loop(0, 16, step4, (lo0, hi0))
    lo, hi = jax.lax.fori_loop(0, 2, step, (lo, hi))
    t = to_f32(lo)
    m = jnp.max(s, axis=1, keepdims=True)
    w = jnp.where(s >= t, jnp.exp(s - m), 0.0)
    denom = jnp.sum(w, axis=1, keepdims=True)
    attn = w / denom
    o_ref[0] = jnp.dot(attn, v, preferred_element_type=jnp.float32)


def _proj_body(x_ref, w_ref, b_ref, out_ref):
    out_ref[...] = (
        jnp.dot(x_ref[...], w_ref[...], preferred_element_type=jnp.float32)
        + b_ref[0][None, :]
    )


def kernel(x, Wq, bq, Wk, bk, Wv, bv, Wo, bo):
    b, s_len, d = x.shape
    h, dh = _H, d // _H
    scale = (dh ** -0.5) / _TEMPERATURE
    x2 = x.reshape(s_len, d)

    w3 = jnp.stack([Wq, Wk, Wv])
    b3 = jnp.stack([bq, bk, bv]).reshape(3, 1, d)

    nq = s_len // _BQ
    qkv = pl.pallas_call(
        _qkv_body,
        grid=(3, nq),
        in_specs=[
            pl.BlockSpec((_BQ, d), lambda j, i: (i, 0)),
            pl.BlockSpec((1, d, d), lambda j, i: (j, 0, 0)),
            pl.BlockSpec((1, 1, d), lambda j, i: (j, 0, 0)),
        ],
        out_specs=pl.BlockSpec((1, _BQ, d), lambda j, i: (j, i, 0)),
        out_shape=jax.ShapeDtypeStruct((3, s_len, d), jnp.float32),
    )(x2, w3, b3)

    q3 = qkv[0].reshape(s_len, h, dh).transpose(1, 0, 2)
    kt3 = qkv[1].reshape(s_len, h, dh).transpose(1, 2, 0)
    v3 = qkv[2].reshape(s_len, h, dh).transpose(1, 0, 2)

    o3 = pl.pallas_call(
        functools.partial(_attn_body, topk=_TOPK, scale=scale),
        grid=(h, nq),
        in_specs=[
            pl.BlockSpec((1, _BQ, dh), lambda hh, i: (hh, i, 0)),
            pl.BlockSpec((1, dh, s_len), lambda hh, i: (hh, 0, 0)),
            pl.BlockSpec((1, s_len, dh), lambda hh, i: (hh, 0, 0)),
        ],
        out_specs=pl.BlockSpec((1, _BQ, dh), lambda hh, i: (hh, i, 0)),
        out_shape=jax.ShapeDtypeStruct((h, s_len, dh), jnp.float32),
    )(q3, kt3, v3)

    o2 = o3.transpose(1, 0, 2).reshape(s_len, d)

    out = pl.pallas_call(
        _proj_body,
        grid=(nq,),
        in_specs=[
            pl.BlockSpec((_BQ, d), lambda i: (i, 0)),
            pl.BlockSpec((d, d), lambda i: (0, 0)),
            pl.BlockSpec((1, d), lambda i: (0, 0)),
        ],
        out_specs=pl.BlockSpec((_BQ, d), lambda i: (i, 0)),
        out_shape=jax.ShapeDtypeStruct((s_len, d), jnp.float32),
    )(o2, Wo, bo.reshape(1, d))

    return out.reshape(b, s_len, d)


# speedup vs baseline: 1.3504x; 1.3504x over previous
import functools
import jax
import jax.numpy as jnp
from jax.experimental import pallas as pl

_H = 16
_TOPK = 64
_TEMPERATURE = 1.0
_BQ = 2048


def _qkv_body(x_ref, w_ref, b_ref, out_ref):
    x = x_ref[...]
    w = w_ref[0]
    b = b_ref[0]
    out_ref[0] = jnp.dot(x, w, preferred_element_type=jnp.float32) + b[0][None, :]


def _attn_body(q_ref, kt_ref, v_ref, o_ref, *, topk, scale):
    q = q_ref[0]
    kt = kt_ref[0]
    v = v_ref[0]
    s = jnp.dot(q, kt, preferred_element_type=jnp.float32) * scale

    def to_f32(k):
        return jax.lax.bitcast_convert_type(
            k ^ ((k >> 31) & jnp.int32(0x7FFFFFFF)), jnp.float32
        )

    bq = s.shape[0]
    lo0 = jnp.full((bq, 1), jnp.int32(-2139095041), jnp.int32)
    hi0 = jnp.full((bq, 1), jnp.int32(0x7F800000), jnp.int32)

    def step(_, carry):
        lo, hi = carry
        mid = (lo >> 1) + (hi >> 1) + (lo & hi & 1)
        cnt = jnp.sum((s >= to_f32(mid)).astype(jnp.int32), axis=1, keepdims=True)
        ge = cnt >= topk
        return jnp.where(ge, mid, lo), jnp.where(ge, hi, mid)

    lo, hi = jax.lax.fori_loop(0, 32, step, (lo0, hi0))
    t = to_f32(lo)
    m = jnp.max(s, axis=1, keepdims=True)
    w = jnp.where(s >= t, jnp.exp(s - m), 0.0)
    denom = jnp.sum(w, axis=1, keepdims=True)
    attn = w / denom
    o_ref[0] = jnp.dot(attn, v, preferred_element_type=jnp.float32)


def _proj_body(x_ref, w_ref, b_ref, out_ref):
    out_ref[...] = (
        jnp.dot(x_ref[...], w_ref[...], preferred_element_type=jnp.float32)
        + b_ref[0][None, :]
    )


def kernel(x, Wq, bq, Wk, bk, Wv, bv, Wo, bo):
    b, s_len, d = x.shape
    h, dh = _H, d // _H
    scale = (dh ** -0.5) / _TEMPERATURE
    x2 = x.reshape(s_len, d)

    w3 = jnp.stack([Wq, Wk, Wv])
    b3 = jnp.stack([bq, bk, bv]).reshape(3, 1, d)

    nq = s_len // _BQ
    qkv = pl.pallas_call(
        _qkv_body,
        grid=(3, nq),
        in_specs=[
            pl.BlockSpec((_BQ, d), lambda j, i: (i, 0)),
            pl.BlockSpec((1, d, d), lambda j, i: (j, 0, 0)),
            pl.BlockSpec((1, 1, d), lambda j, i: (j, 0, 0)),
        ],
        out_specs=pl.BlockSpec((1, _BQ, d), lambda j, i: (j, i, 0)),
        out_shape=jax.ShapeDtypeStruct((3, s_len, d), jnp.float32),
    )(x2, w3, b3)

    q3 = qkv[0].reshape(s_len, h, dh).transpose(1, 0, 2)
    kt3 = qkv[1].reshape(s_len, h, dh).transpose(1, 2, 0)
    v3 = qkv[2].reshape(s_len, h, dh).transpose(1, 0, 2)

    o3 = pl.pallas_call(
        functools.partial(_attn_body, topk=_TOPK, scale=scale),
        grid=(h, nq),
        in_specs=[
            pl.BlockSpec((1, _BQ, dh), lambda hh, i: (hh, i, 0)),
            pl.BlockSpec((1, dh, s_len), lambda hh, i: (hh, 0, 0)),
            pl.BlockSpec((1, s_len, dh), lambda hh, i: (hh, 0, 0)),
        ],
        out_specs=pl.BlockSpec((1, _BQ, dh), lambda hh, i: (hh, i, 0)),
        out_shape=jax.ShapeDtypeStruct((h, s_len, dh), jnp.float32),
    )(q3, kt3, v3)

    o2 = o3.transpose(1, 0, 2).reshape(s_len, d)

    out = pl.pallas_call(
        _proj_body,
        grid=(nq,),
        in_specs=[
            pl.BlockSpec((_BQ, d), lambda i: (i, 0)),
            pl.BlockSpec((d, d), lambda i: (0, 0)),
            pl.BlockSpec((1, d), lambda i: (0, 0)),
        ],
        out_specs=pl.BlockSpec((_BQ, d), lambda i: (i, 0)),
        out_shape=jax.ShapeDtypeStruct((s_len, d), jnp.float32),
    )(o2, Wo, bo.reshape(1, d))

    return out.reshape(b, s_len, d)
